# baseline (speedup 1.0000x reference)
import functools

import jax
import jax.numpy as jnp
from jax import lax
from jax.experimental import pallas as pl
from jax.experimental.pallas import tpu as pltpu
from jax.experimental.pallas import tpu_sc as plsc

_L = 16


def _make_lookup(n_elems, n_workers, chunk):
    per_w = n_elems // n_workers
    n_chunks = per_w // chunk
    mesh = plsc.VectorSubcoreMesh(core_axis_name="c", subcore_axis_name="s")

    @functools.partial(
        pl.kernel,
        mesh=mesh,
        out_type=jax.ShapeDtypeStruct((n_elems,), jnp.int32),
        scratch_types=[
            pltpu.VMEM((_L,), jnp.int32),
            pltpu.VMEM((chunk,), jnp.int32),
            pltpu.VMEM((chunk,), jnp.int32),
        ],
        compiler_params=pltpu.CompilerParams(needs_layout_passes=False),
    )
    def lookup(idx_hbm, vals_hbm, out_hbm, tab_v, in_v, out_v):
        wid = lax.axis_index("s") * 2 + lax.axis_index("c")
        pltpu.sync_copy(vals_hbm, tab_v)
        base = wid * per_w

        def chunk_body(ci, _):
            off = base + ci * chunk
            pltpu.sync_copy(idx_hbm.at[pl.ds(off, chunk)], in_v)

            def vec_body(i, _):
                s = pl.ds(i * _L, _L)
                out_v[s] = plsc.load_gather(tab_v, [in_v[s]])
                return 0

            lax.fori_loop(0, chunk // _L, vec_body, 0)
            pltpu.sync_copy(out_v, out_hbm.at[pl.ds(off, chunk)])
            return 0

        lax.fori_loop(0, n_chunks, chunk_body, 0)

    return lookup


def kernel(inputs, vals):
    shape = inputs.shape
    x = inputs.reshape(-1).astype(jnp.int32)
    tab = jnp.pad(vals.astype(jnp.int32), (0, _L - vals.shape[0]))
    out = _make_lookup(x.shape[0], 32, 20480)(x, tab)
    return out.reshape(shape)


# speedup vs baseline: 1.3557x; 1.3557x over previous
import functools

import jax
import jax.numpy as jnp
from jax import lax
from jax.experimental import pallas as pl
from jax.experimental.pallas import tpu as pltpu
from jax.experimental.pallas import tpu_sc as plsc

_L = 16


def _make_lookup(n_elems, n_workers, chunk):
    per_w = n_elems // n_workers
    n_chunks = per_w // chunk
    mesh = plsc.VectorSubcoreMesh(core_axis_name="c", subcore_axis_name="s")

    @functools.partial(
        pl.kernel,
        mesh=mesh,
        out_type=jax.ShapeDtypeStruct((n_elems,), jnp.int32),
        scratch_types=[
            pltpu.VMEM((_L,), jnp.int32),
            [pltpu.VMEM((chunk,), jnp.int32) for _ in range(2)],
            [pltpu.VMEM((chunk,), jnp.int32) for _ in range(2)],
            [pltpu.SemaphoreType.DMA for _ in range(2)],
            [pltpu.SemaphoreType.DMA for _ in range(2)],
        ],
        compiler_params=pltpu.CompilerParams(needs_layout_passes=False),
    )
    def lookup(idx_hbm, vals_hbm, out_hbm, tab_v, in_b, out_b, in_sem, out_sem):
        wid = lax.axis_index("s") * 2 + lax.axis_index("c")
        pltpu.sync_copy(vals_hbm, tab_v)
        base = wid * per_w

        def compute(src, dst):
            @plsc.parallel_loop(0, chunk // _L, unroll=8)
            def _(i):
                s = pl.ds(i * _L, _L)
                dst[s] = plsc.load_gather(tab_v, [src[s]])

        in_cp = [None, None]
        out_cp = [None, None]
        in_cp[0] = pltpu.async_copy(
            idx_hbm.at[pl.ds(base, chunk)], in_b[0], in_sem[0])
        for c in range(n_chunks):
            b = c % 2
            if c + 1 < n_chunks:
                nb = (c + 1) % 2
                in_cp[nb] = pltpu.async_copy(
                    idx_hbm.at[pl.ds(base + (c + 1) * chunk, chunk)],
                    in_b[nb], in_sem[nb])
            in_cp[b].wait()
            if c >= 2:
                out_cp[b].wait()
            compute(in_b[b], out_b[b])
            out_cp[b] = pltpu.async_copy(
                out_b[b], out_hbm.at[pl.ds(base + c * chunk, chunk)],
                out_sem[b])
        for c in (n_chunks - 2, n_chunks - 1):
            if c >= 0:
                out_cp[c % 2].wait()

    return lookup


def kernel(inputs, vals):
    shape = inputs.shape
    x = inputs.reshape(-1).astype(jnp.int32)
    tab = jnp.pad(vals.astype(jnp.int32), (0, _L - vals.shape[0]))
    out = _make_lookup(x.shape[0], 32, 20480)(x, tab)
    return out.reshape(shape)
